# baseline (speedup 1.0000x reference)
import jax
import jax.numpy as jnp
from jax import lax
from jax.experimental import pallas as pl
from jax.experimental.pallas import tpu as pltpu


def _pos_body(row_ref, col_ref, out_ref, pos_scratch):
    i = pl.program_id(0)

    @pl.when(i == 0)
    def _():
        p = lax.broadcasted_iota(jnp.int32, (32, 1024), 1)
        g = lax.broadcasted_iota(jnp.int32, (32, 1024), 0)
        sel_w = (p % 32 == g).astype(jnp.float32)
        sel_h = (p // 32 == g).astype(jnp.float32)
        dn = (((0,), (0,)), ((), ()))
        top = lax.dot_general(
            col_ref[0:32, :], sel_w, dn, preferred_element_type=jnp.float32)
        bot = lax.dot_general(
            row_ref[0:32, :], sel_h, dn, preferred_element_type=jnp.float32)
        pos_scratch[0:256] = top.reshape(256, 32, 32)
        pos_scratch[256:512] = bot.reshape(256, 32, 32)

    out_ref[0] = pos_scratch[...]


def kernel(x, row_embed, col_embed):
    b = x.shape[0]
    return pl.pallas_call(
        _pos_body,
        grid=(b,),
        in_specs=[
            pl.BlockSpec((50, 256), lambda i: (0, 0)),
            pl.BlockSpec((50, 256), lambda i: (0, 0)),
        ],
        out_specs=pl.BlockSpec((1, 512, 32, 32), lambda i: (i, 0, 0, 0)),
        out_shape=jax.ShapeDtypeStruct((b, 512, 32, 32), jnp.float32),
        scratch_shapes=[pltpu.VMEM((512, 32, 32), jnp.float32)],
    )(row_embed, col_embed)


# speedup vs baseline: 2.6964x; 2.6964x over previous
import jax
import jax.numpy as jnp
from jax import lax
from jax.experimental import pallas as pl
from jax.experimental.pallas import tpu as pltpu


def _pos_body(row_ref, col_ref, out_ref, pos_scratch):
    i = pl.program_id(0)

    @pl.when(i == 0)
    def _():
        p = lax.broadcasted_iota(jnp.int32, (32, 1024), 1)
        g = lax.broadcasted_iota(jnp.int32, (32, 1024), 0)
        sel_w = (p % 32 == g).astype(jnp.float32)
        sel_h = (p // 32 == g).astype(jnp.float32)
        dn = (((0,), (0,)), ((), ()))
        top = lax.dot_general(
            col_ref[0:32, :], sel_w, dn, preferred_element_type=jnp.float32)
        bot = lax.dot_general(
            row_ref[0:32, :], sel_h, dn, preferred_element_type=jnp.float32)
        pos_scratch[0:256] = top
        pos_scratch[256:512] = bot

    out_ref[0] = pos_scratch[...]


def kernel(x, row_embed, col_embed):
    b = x.shape[0]
    out = pl.pallas_call(
        _pos_body,
        grid=(b,),
        in_specs=[
            pl.BlockSpec((50, 256), lambda i: (0, 0)),
            pl.BlockSpec((50, 256), lambda i: (0, 0)),
        ],
        out_specs=pl.BlockSpec((1, 512, 1024), lambda i: (i, 0, 0)),
        out_shape=jax.ShapeDtypeStruct((b, 512, 1024), jnp.float32),
        scratch_shapes=[pltpu.VMEM((512, 1024), jnp.float32)],
    )(row_embed, col_embed)
    return out.reshape(b, 512, 32, 32)


# speedup vs baseline: 8.7727x; 3.2535x over previous
import jax
import jax.numpy as jnp
from jax import lax
from jax.experimental import pallas as pl
from jax.experimental.pallas import tpu as pltpu


def _pos_body(row_ref, col_ref, out_ref):
    col32 = col_ref[0:32, :]
    row32 = row_ref[0:32, :]
    left = jnp.broadcast_to(col32[None, :, :], (32, 32, 256)).reshape(1024, 256)
    right = jnp.broadcast_to(row32[:, None, :], (32, 32, 256)).reshape(1024, 256)
    out_ref[0, :, 0:256] = left
    out_ref[0, :, 256:512] = right


def kernel(x, row_embed, col_embed):
    b = x.shape[0]
    out = pl.pallas_call(
        _pos_body,
        grid=(b,),
        in_specs=[
            pl.BlockSpec((50, 256), lambda i: (0, 0)),
            pl.BlockSpec((50, 256), lambda i: (0, 0)),
        ],
        out_specs=pl.BlockSpec((1, 1024, 512), lambda i: (i, 0, 0)),
        out_shape=jax.ShapeDtypeStruct((b, 1024, 512), jnp.float32),
    )(row_embed, col_embed)
    return jnp.transpose(out.reshape(b, 32, 32, 512), (0, 3, 1, 2))


# speedup vs baseline: 9.6509x; 1.1001x over previous
import jax
import jax.numpy as jnp
from jax import lax
from jax.experimental import pallas as pl
from jax.experimental.pallas import tpu as pltpu


def _pos_body(row_ref, col_ref, out_ref):
    col32 = col_ref[0:32, :]
    row32 = row_ref[0:32, :]
    left = jnp.broadcast_to(col32[None, :, :], (32, 32, 256)).reshape(1024, 256)
    right = jnp.broadcast_to(row32[:, None, :], (32, 32, 256)).reshape(1024, 256)
    out_ref[0, :, 0:256] = left
    out_ref[0, :, 256:512] = right
    out_ref[1, :, 0:256] = left
    out_ref[1, :, 256:512] = right


def kernel(x, row_embed, col_embed):
    b = x.shape[0]
    out = pl.pallas_call(
        _pos_body,
        grid=(b // 2,),
        in_specs=[
            pl.BlockSpec((50, 256), lambda i: (0, 0)),
            pl.BlockSpec((50, 256), lambda i: (0, 0)),
        ],
        out_specs=pl.BlockSpec((2, 1024, 512), lambda i: (i, 0, 0)),
        out_shape=jax.ShapeDtypeStruct((b, 1024, 512), jnp.float32),
    )(row_embed, col_embed)
    return jnp.transpose(out.reshape(b, 32, 32, 512), (0, 3, 1, 2))
